# baseline (speedup 1.0000x reference)
import functools

import jax
import jax.numpy as jnp
from jax import lax
from jax.experimental import pallas as pl
from jax.experimental.pallas import tpu as pltpu
from jax.experimental.pallas import tpu_sc as plsc

N = 100000
E = 3200000
CH = 8

NC = 2
NS = 16
NW = NC * NS
EP = 3211264
NPIECE = EP // 128
EW = EP // NW
CEG = 2048
NCHG = EW // CEG
NQ = 4
EQ = EP // NQ
CES = 2048
NCHS = EQ // CES

BE = 8192
NB = 2048
NP = 100096


def _sc_mesh():
    return plsc.VectorSubcoreMesh(core_axis_name="c", subcore_axis_name="s")


_SC_PARAMS = pltpu.CompilerParams(
    use_tc_tiling_on_sc=False, needs_layout_passes=False
)


def _gather_body(x8_hbm, dst_hbm, src_hbm, xdp_hbm, xmp_hbm,
                 ib0_d, ib0_s, ib1_d, ib1_s,
                 rows0_d, rows0_s, rows1_d, rows1_s,
                 tbuf_d, tbuf_m, sem_i, sem_g, sem_o):
    wid = lax.axis_index("s") * NC + lax.axis_index("c")
    base0 = wid * EW
    lanes = lax.iota(jnp.int32, 16)

    def fire_idx(kc, ib_d, ib_s):
        b = base0 + kc * CEG
        pltpu.async_copy(dst_hbm.at[pl.ds(b, CEG)], ib_d, sem_i)
        pltpu.async_copy(src_hbm.at[pl.ds(b, CEG)], ib_s, sem_i)

    def drain_idx(ib_d, ib_s):
        pltpu.make_async_copy(dst_hbm.at[pl.ds(0, CEG)], ib_d, sem_i).wait()
        pltpu.make_async_copy(src_hbm.at[pl.ds(0, CEG)], ib_s, sem_i).wait()

    def fire_gathers(ib, rows):
        for j in range(CEG // 128):
            pltpu.async_copy(
                x8_hbm.at[ib.at[pl.ds(j * 128, 128)]],
                rows.at[pl.ds(j * 128, 128)], sem_g)

    def drain_gathers(rows):
        pltpu.make_async_copy(x8_hbm.at[pl.ds(0, CEG)], rows, sem_g).wait()

    fire_idx(0, ib0_d, ib0_s)
    drain_idx(ib0_d, ib0_s)
    fire_gathers(ib0_d, rows0_d)
    fire_gathers(ib0_s, rows0_s)

    def do_chunk(k, bufs, nbufs):
        ib_d, ib_s, rows_d, rows_s = bufs
        nib_d, nib_s, nrows_d, nrows_s = nbufs
        kn = jnp.minimum(k + 1, NCHG - 1)
        fire_idx(kn, nib_d, nib_s)
        drain_gathers(rows_d)
        drain_gathers(rows_s)
        drain_idx(nib_d, nib_s)
        fire_gathers(nib_d, nrows_d)
        fire_gathers(nib_s, nrows_s)
        @plsc.parallel_loop(0, CEG // 16, unroll=2)
        def tgrp(g):
            ridx = lanes + g * 16
            pe = g // 8
            off = (g % 8) * 16
            for c in range(CH):
                csp = jnp.full((16,), c, jnp.int32)
                a = plsc.load_gather(rows_d, [ridx, csp])
                b = plsc.load_gather(rows_s, [ridx, csp])
                tbuf_d[pe, c, pl.ds(off, 16)] = a
                tbuf_m[pe, c, pl.ds(off, 16)] = b - a
        pb = (base0 + k * CEG) // 128
        pltpu.sync_copy(tbuf_d, xdp_hbm.at[pl.ds(pb, CEG // 128)])
        pltpu.sync_copy(tbuf_m, xmp_hbm.at[pl.ds(pb, CEG // 128)])

    bufs0 = (ib0_d, ib0_s, rows0_d, rows0_s)
    bufs1 = (ib1_d, ib1_s, rows1_d, rows1_s)

    def pair(jj, carry):
        do_chunk(jj * 2, bufs0, bufs1)
        do_chunk(jj * 2 + 1, bufs1, bufs0)
        return carry

    lax.fori_loop(0, NCHG // 2, pair, 0)
    do_chunk(NCHG - 1, bufs0, bufs1)
    drain_gathers(rows1_d)
    drain_gathers(rows1_s)


def _sc_gather(x8, dst, src):
    k = functools.partial(
        pl.kernel,
        out_type=(
            jax.ShapeDtypeStruct((NPIECE, CH, 128), jnp.float32),
            jax.ShapeDtypeStruct((NPIECE, CH, 128), jnp.float32),
        ),
        mesh=_sc_mesh(),
        scratch_types=[
            pltpu.VMEM((CEG,), jnp.int32),
            pltpu.VMEM((CEG,), jnp.int32),
            pltpu.VMEM((CEG,), jnp.int32),
            pltpu.VMEM((CEG,), jnp.int32),
            pltpu.VMEM((CEG, CH), jnp.float32),
            pltpu.VMEM((CEG, CH), jnp.float32),
            pltpu.VMEM((CEG, CH), jnp.float32),
            pltpu.VMEM((CEG, CH), jnp.float32),
            pltpu.VMEM((CEG // 128, CH, 128), jnp.float32),
            pltpu.VMEM((CEG // 128, CH, 128), jnp.float32),
            pltpu.SemaphoreType.DMA,
            pltpu.SemaphoreType.DMA,
            pltpu.SemaphoreType.DMA,
        ],
        compiler_params=_SC_PARAMS,
    )(_gather_body)
    return k(x8, dst, src)


def _scatter_body_full(msgp_hbm, dst_hbm, part_hbm, acc_v,
                       ib0, vb0, ib1, vb1, sem_i, sem_v):
    wid = lax.axis_index("s") * NC + lax.axis_index("c")
    ch = wid // NQ
    q = wid % NQ

    minf = jnp.full((16,), -jnp.inf, jnp.float32)

    @plsc.parallel_loop(0, NP // 16, unroll=4)
    def initb(i):
        acc_v[pl.ds(i * 16, 16)] = minf

    base0 = q * EQ

    def fire(kc, ib, vb):
        b = base0 + kc * CES
        pltpu.async_copy(dst_hbm.at[pl.ds(b, CES)], ib, sem_i)
        pltpu.async_copy(
            msgp_hbm.at[pl.ds(b // 128, CES // 128), ch], vb, sem_v)

    def drain(ib, vb):
        pltpu.make_async_copy(dst_hbm.at[pl.ds(0, CES)], ib, sem_i).wait()
        pltpu.make_async_copy(
            msgp_hbm.at[pl.ds(0, CES // 128), 0], vb, sem_v).wait()

    fire(0, ib0, vb0)

    def do_chunk(k, ib, vb, nib, nvb):
        kn = jnp.minimum(k + 1, NCHS - 1)
        fire(kn, nib, nvb)
        drain(ib, vb)

        @plsc.parallel_loop(0, CES // 16, unroll=4)
        def grp_fast(g):
            idx = ib[pl.ds(g * 16, 16)]
            val = vb[g // 8, pl.ds((g % 8) * 16, 16)]
            cur = plsc.load_gather(acc_v, [idx])
            plsc.store_scatter(acc_v, [idx], jnp.maximum(cur, val))

        @plsc.parallel_loop(
            0, CES // 16, unroll=4, carry=jnp.zeros((16,), jnp.bool_))
        def flag(g, fl):
            idx = ib[pl.ds(g * 16, 16)]
            val = vb[g // 8, pl.ds((g % 8) * 16, 16)]
            got = plsc.load_gather(acc_v, [idx])
            return fl | (val > got)

        @pl.when(jnp.any(flag))
        def _():
            def grp_slow(g, carry):
                idx = ib[pl.ds(g * 16, 16)]
                val = vb[g // 8, pl.ds((g % 8) * 16, 16)]
                cur = plsc.load_gather(acc_v, [idx])
                plsc.store_scatter(acc_v, [idx], jnp.maximum(cur, val))
                got = plsc.load_gather(acc_v, [idx])
                pend = val > got

                @pl.when(jnp.any(pend))
                def _():
                    def fix(i, s):
                        got_, p_ = s
                        plsc.store_scatter(
                            acc_v, [idx], jnp.maximum(got_, val), mask=p_)
                        g2 = plsc.load_gather(acc_v, [idx])
                        return g2, val > g2

                    lax.fori_loop(0, 15, fix, (got, pend))

                return carry

            lax.fori_loop(0, CES // 16, grp_slow, 0)

    def pair(jj, carry):
        do_chunk(jj * 2, ib0, vb0, ib1, vb1)
        do_chunk(jj * 2 + 1, ib1, vb1, ib0, vb0)
        return carry

    lax.fori_loop(0, NCHS // 2, pair, 0)
    drain(ib0, vb0)
    pltpu.sync_copy(acc_v, part_hbm.at[q, ch])


def _sc_scatter(msgp, dst):
    k = functools.partial(
        pl.kernel,
        out_type=jax.ShapeDtypeStruct((NQ, CH, NP), jnp.float32),
        mesh=_sc_mesh(),
        scratch_types=[
            pltpu.VMEM((NP,), jnp.float32),
            pltpu.VMEM((CES,), jnp.int32),
            pltpu.VMEM((CES // 128, 128), jnp.float32),
            pltpu.VMEM((CES,), jnp.int32),
            pltpu.VMEM((CES // 128, 128), jnp.float32),
            pltpu.SemaphoreType.DMA,
            pltpu.SemaphoreType.DMA,
        ],
        compiler_params=_SC_PARAMS,
    )(_scatter_body_full)
    return k(msgp, dst)


def _enc_body(x_ref, w_ref, b_ref, out_ref):
    out_ref[...] = x_ref[...] @ w_ref[...] + b_ref[...]


def _tc_encode(x, enc_W, enc_b):
    grid = (NP + NB - 1) // NB
    return pl.pallas_call(
        _enc_body,
        grid=(grid,),
        in_specs=[
            pl.BlockSpec((NB, 3), lambda i: (i, 0)),
            pl.BlockSpec((3, CH), lambda i: (0, 0)),
            pl.BlockSpec((1, CH), lambda i: (0, 0)),
        ],
        out_specs=pl.BlockSpec((NB, CH), lambda i: (i, 0)),
        out_shape=jax.ShapeDtypeStruct((NP, CH), jnp.float32),
    )(x, enc_W, enc_b.reshape(1, CH))


def _mlp_body(xdp_ref, xmp_ref, w0t_ref, b0_ref, w1t_ref, b1_ref, w2t_ref,
              b2_ref, out_ref):
    npc = BE // 128
    xdp = xdp_ref[...]
    xmp = xmp_ref[...]
    xd = jnp.concatenate([xdp[j] for j in range(npc)], axis=1)
    xm = jnp.concatenate([xmp[j] for j in range(npc)], axis=1)
    h = jnp.concatenate([xd, xm], axis=0)
    h = jnp.maximum(w0t_ref[...] @ h + b0_ref[...], 0.0)
    h = jnp.maximum(w1t_ref[...] @ h + b1_ref[...], 0.0)
    m = w2t_ref[...] @ h + b2_ref[...]
    out_ref[...] = jnp.stack(
        [m[:, 128 * j:128 * (j + 1)] for j in range(npc)], axis=0)


def _tc_edge_mlp(xdp, xmp, W0, b0, W1, b1, W2, b2):
    npc = BE // 128
    grid = EP // BE
    return pl.pallas_call(
        _mlp_body,
        grid=(grid,),
        in_specs=[
            pl.BlockSpec((npc, CH, 128), lambda i: (i, 0, 0)),
            pl.BlockSpec((npc, CH, 128), lambda i: (i, 0, 0)),
            pl.BlockSpec((32, 2 * CH), lambda i: (0, 0)),
            pl.BlockSpec((32, 1), lambda i: (0, 0)),
            pl.BlockSpec((32, 32), lambda i: (0, 0)),
            pl.BlockSpec((32, 1), lambda i: (0, 0)),
            pl.BlockSpec((CH, 32), lambda i: (0, 0)),
            pl.BlockSpec((CH, 1), lambda i: (0, 0)),
        ],
        out_specs=pl.BlockSpec((npc, CH, 128), lambda i: (i, 0, 0)),
        out_shape=jax.ShapeDtypeStruct((NPIECE, CH, 128), jnp.float32),
    )(xdp, xmp, W0.T, b0.reshape(32, 1), W1.T, b1.reshape(32, 1),
      W2.T, b2.reshape(CH, 1))


def _combine_body(part_ref, out_ref):
    p = part_ref[...]
    m = jnp.max(p, axis=0)
    m = jnp.where(jnp.isneginf(m), 0.0, m)
    m = jnp.maximum(m, 0.0)
    out_ref[...] = m.T


def _tc_combine(part):
    grid = (NP + NB - 1) // NB
    return pl.pallas_call(
        _combine_body,
        grid=(grid,),
        in_specs=[pl.BlockSpec((NQ, CH, NB), lambda i: (0, 0, i))],
        out_specs=pl.BlockSpec((NB, CH), lambda i: (i, 0)),
        out_shape=jax.ShapeDtypeStruct((NP, CH), jnp.float32),
    )(part)


def _head_body(part_ref, w0_ref, b0_ref, w1_ref, b1_ref, w2_ref, b2_ref,
               out_ref):
    p = part_ref[...]
    m = jnp.max(p, axis=0)
    m = jnp.where(jnp.isneginf(m), 0.0, m)
    m = jnp.maximum(m, 0.0)
    xb = m.T
    h = jnp.maximum(xb @ w0_ref[...] + b0_ref[...], 0.0)
    h = jnp.maximum(h @ w1_ref[...] + b1_ref[...], 0.0)
    out_ref[...] = h @ w2_ref[...] + b2_ref[...]


def _tc_head(part, W0, b0, W1, b1, W2, b2):
    grid = (N + NB - 1) // NB
    return pl.pallas_call(
        _head_body,
        grid=(grid,),
        in_specs=[
            pl.BlockSpec((NQ, CH, NB), lambda i: (0, 0, i)),
            pl.BlockSpec((CH, 128), lambda i: (0, 0)),
            pl.BlockSpec((1, 128), lambda i: (0, 0)),
            pl.BlockSpec((128, 128), lambda i: (0, 0)),
            pl.BlockSpec((1, 128), lambda i: (0, 0)),
            pl.BlockSpec((128, 1), lambda i: (0, 0)),
            pl.BlockSpec((1, 1), lambda i: (0, 0)),
        ],
        out_specs=pl.BlockSpec((NB, 1), lambda i: (i, 0)),
        out_shape=jax.ShapeDtypeStruct((N, 1), jnp.float32),
    )(part, W0, b0.reshape(1, 128), W1, b1.reshape(1, 128),
      W2, b2.reshape(1, 1))


def kernel(x, edge_index, enc_W, enc_b,
           c0_W0, c0_b0, c0_W1, c0_b1, c0_W2, c0_b2,
           c1_W0, c1_b0, c1_W1, c1_b1, c1_W2, c1_b2,
           m_W0, m_b0, m_W1, m_b1, m_W2, m_b2):
    src = edge_index[0].astype(jnp.int32)
    dst = edge_index[1].astype(jnp.int32)
    pad = EP - E
    src = jnp.concatenate([src, jnp.zeros((pad,), jnp.int32)])
    dst = jnp.concatenate([dst, jnp.full((pad,), N, jnp.int32)])

    X = _tc_encode(x, enc_W, enc_b)

    xdp, xmp = _sc_gather(X, dst, src)
    msgp = _tc_edge_mlp(xdp, xmp, c0_W0, c0_b0, c0_W1, c0_b1, c0_W2, c0_b2)
    part = _sc_scatter(msgp, dst)
    X = _tc_combine(part)

    xdp, xmp = _sc_gather(X, dst, src)
    msgp = _tc_edge_mlp(xdp, xmp, c1_W0, c1_b0, c1_W1, c1_b1, c1_W2, c1_b2)
    part = _sc_scatter(msgp, dst)

    return _tc_head(part, m_W0, m_b0, m_W1, m_b1, m_W2, m_b2)


# speedup vs baseline: 1.3829x; 1.3829x over previous
import functools

import jax
import jax.numpy as jnp
from jax import lax
from jax.experimental import pallas as pl
from jax.experimental.pallas import tpu as pltpu
from jax.experimental.pallas import tpu_sc as plsc

N = 100000
E = 3200000
CH = 8

NC = 2
NS = 16
NW = NC * NS
EP = 3211264
NPIECE = EP // 128
EW = EP // NW
CEG = 2048
NCHG = EW // CEG
NQ = 4
EQ = EP // NQ
CES = 4096
NCHS = EQ // CES

BE = 8192
NB = 2048
NP = 100096


def _sc_mesh():
    return plsc.VectorSubcoreMesh(core_axis_name="c", subcore_axis_name="s")


_SC_PARAMS = pltpu.CompilerParams(
    use_tc_tiling_on_sc=False, needs_layout_passes=False
)


def _gather_body(x8_hbm, dst_hbm, src_hbm, xdp_hbm, xmp_hbm,
                 ib0_d, ib0_s, ib1_d, ib1_s,
                 rows0_d, rows0_s, rows1_d, rows1_s,
                 tbuf_d, tbuf_m, sem_i, sem_g, sem_o):
    wid = lax.axis_index("s") * NC + lax.axis_index("c")
    base0 = wid * EW
    lanes = lax.iota(jnp.int32, 16)

    def fire_idx(kc, ib_d, ib_s):
        b = base0 + kc * CEG
        pltpu.async_copy(dst_hbm.at[pl.ds(b, CEG)], ib_d, sem_i)
        pltpu.async_copy(src_hbm.at[pl.ds(b, CEG)], ib_s, sem_i)

    def drain_idx(ib_d, ib_s):
        pltpu.make_async_copy(dst_hbm.at[pl.ds(0, CEG)], ib_d, sem_i).wait()
        pltpu.make_async_copy(src_hbm.at[pl.ds(0, CEG)], ib_s, sem_i).wait()

    def fire_gathers(ib, rows):
        for j in range(CEG // 128):
            pltpu.async_copy(
                x8_hbm.at[ib.at[pl.ds(j * 128, 128)]],
                rows.at[pl.ds(j * 128, 128)], sem_g)

    def drain_gathers(rows):
        pltpu.make_async_copy(x8_hbm.at[pl.ds(0, CEG)], rows, sem_g).wait()

    fire_idx(0, ib0_d, ib0_s)
    drain_idx(ib0_d, ib0_s)
    fire_gathers(ib0_d, rows0_d)
    fire_gathers(ib0_s, rows0_s)

    def do_chunk(k, bufs, nbufs):
        ib_d, ib_s, rows_d, rows_s = bufs
        nib_d, nib_s, nrows_d, nrows_s = nbufs
        kn = jnp.minimum(k + 1, NCHG - 1)
        fire_idx(kn, nib_d, nib_s)
        drain_gathers(rows_d)
        drain_gathers(rows_s)
        drain_idx(nib_d, nib_s)
        fire_gathers(nib_d, nrows_d)
        fire_gathers(nib_s, nrows_s)
        @plsc.parallel_loop(0, CEG // 16, unroll=2)
        def tgrp(g):
            ridx = lanes + g * 16
            pe = g // 8
            off = (g % 8) * 16
            for c in range(CH):
                csp = jnp.full((16,), c, jnp.int32)
                a = plsc.load_gather(rows_d, [ridx, csp])
                b = plsc.load_gather(rows_s, [ridx, csp])
                tbuf_d[pe, c, pl.ds(off, 16)] = a
                tbuf_m[pe, c, pl.ds(off, 16)] = b - a
        pb = (base0 + k * CEG) // 128
        pltpu.sync_copy(tbuf_d, xdp_hbm.at[pl.ds(pb, CEG // 128)])
        pltpu.sync_copy(tbuf_m, xmp_hbm.at[pl.ds(pb, CEG // 128)])

    bufs0 = (ib0_d, ib0_s, rows0_d, rows0_s)
    bufs1 = (ib1_d, ib1_s, rows1_d, rows1_s)

    def pair(jj, carry):
        do_chunk(jj * 2, bufs0, bufs1)
        do_chunk(jj * 2 + 1, bufs1, bufs0)
        return carry

    lax.fori_loop(0, NCHG // 2, pair, 0)
    do_chunk(NCHG - 1, bufs0, bufs1)
    drain_gathers(rows1_d)
    drain_gathers(rows1_s)


def _sc_gather(x8, dst, src):
    k = functools.partial(
        pl.kernel,
        out_type=(
            jax.ShapeDtypeStruct((NPIECE, CH, 128), jnp.float32),
            jax.ShapeDtypeStruct((NPIECE, CH, 128), jnp.float32),
        ),
        mesh=_sc_mesh(),
        scratch_types=[
            pltpu.VMEM((CEG,), jnp.int32),
            pltpu.VMEM((CEG,), jnp.int32),
            pltpu.VMEM((CEG,), jnp.int32),
            pltpu.VMEM((CEG,), jnp.int32),
            pltpu.VMEM((CEG, CH), jnp.float32),
            pltpu.VMEM((CEG, CH), jnp.float32),
            pltpu.VMEM((CEG, CH), jnp.float32),
            pltpu.VMEM((CEG, CH), jnp.float32),
            pltpu.VMEM((CEG // 128, CH, 128), jnp.float32),
            pltpu.VMEM((CEG // 128, CH, 128), jnp.float32),
            pltpu.SemaphoreType.DMA,
            pltpu.SemaphoreType.DMA,
            pltpu.SemaphoreType.DMA,
        ],
        compiler_params=_SC_PARAMS,
    )(_gather_body)
    return k(x8, dst, src)


def _scatter_body_full(msgp_hbm, dst_hbm, part_hbm, acc_v,
                       ib0, vb0, ib1, vb1, sem_i, sem_v):
    wid = lax.axis_index("s") * NC + lax.axis_index("c")
    ch = wid // NQ
    q = wid % NQ

    minf = jnp.full((16,), -jnp.inf, jnp.float32)

    @plsc.parallel_loop(0, NP // 16, unroll=4)
    def initb(i):
        acc_v[pl.ds(i * 16, 16)] = minf

    base0 = q * EQ

    def fire(kc, ib, vb):
        b = base0 + kc * CES
        pltpu.async_copy(dst_hbm.at[pl.ds(b, CES)], ib, sem_i)
        pltpu.async_copy(
            msgp_hbm.at[pl.ds(b // 128, CES // 128), ch], vb, sem_v)

    def drain(ib, vb):
        pltpu.make_async_copy(dst_hbm.at[pl.ds(0, CES)], ib, sem_i).wait()
        pltpu.make_async_copy(
            msgp_hbm.at[pl.ds(0, CES // 128), 0], vb, sem_v).wait()

    fire(0, ib0, vb0)

    def do_chunk(k, ib, vb, nib, nvb):
        kn = jnp.minimum(k + 1, NCHS - 1)
        fire(kn, nib, nvb)
        drain(ib, vb)

        def grp_fast(g, carry):
            idx = ib[pl.ds(g * 16, 16)]
            val = vb[g // 8, pl.ds((g % 8) * 16, 16)]
            cur = plsc.load_gather(acc_v, [idx])
            plsc.store_scatter(acc_v, [idx], jnp.maximum(cur, val))
            return carry

        lax.fori_loop(0, CES // 16, grp_fast, 0, unroll=4)

        @plsc.parallel_loop(
            0, CES // 16, unroll=4, carry=jnp.zeros((16,), jnp.bool_))
        def flag(g, fl):
            idx = ib[pl.ds(g * 16, 16)]
            val = vb[g // 8, pl.ds((g % 8) * 16, 16)]
            got = plsc.load_gather(acc_v, [idx])
            return fl | (val > got)

        @pl.when(jnp.any(flag))
        def _():
            def grp_slow(g, carry):
                idx = ib[pl.ds(g * 16, 16)]
                val = vb[g // 8, pl.ds((g % 8) * 16, 16)]
                cur = plsc.load_gather(acc_v, [idx])
                plsc.store_scatter(acc_v, [idx], jnp.maximum(cur, val))
                got = plsc.load_gather(acc_v, [idx])
                pend = val > got

                @pl.when(jnp.any(pend))
                def _():
                    def fix(i, s):
                        got_, p_ = s
                        plsc.store_scatter(
                            acc_v, [idx], jnp.maximum(got_, val), mask=p_)
                        g2 = plsc.load_gather(acc_v, [idx])
                        return g2, val > g2

                    lax.fori_loop(0, 15, fix, (got, pend))

                return carry

            lax.fori_loop(0, CES // 16, grp_slow, 0)

    def pair(jj, carry):
        do_chunk(jj * 2, ib0, vb0, ib1, vb1)
        do_chunk(jj * 2 + 1, ib1, vb1, ib0, vb0)
        return carry

    lax.fori_loop(0, NCHS // 2, pair, 0)
    drain(ib0, vb0)
    pltpu.sync_copy(acc_v, part_hbm.at[q, ch])


def _sc_scatter(msgp, dst):
    k = functools.partial(
        pl.kernel,
        out_type=jax.ShapeDtypeStruct((NQ, CH, NP), jnp.float32),
        mesh=_sc_mesh(),
        scratch_types=[
            pltpu.VMEM((NP,), jnp.float32),
            pltpu.VMEM((CES,), jnp.int32),
            pltpu.VMEM((CES // 128, 128), jnp.float32),
            pltpu.VMEM((CES,), jnp.int32),
            pltpu.VMEM((CES // 128, 128), jnp.float32),
            pltpu.SemaphoreType.DMA,
            pltpu.SemaphoreType.DMA,
        ],
        compiler_params=_SC_PARAMS,
    )(_scatter_body_full)
    return k(msgp, dst)


def _enc_body(x_ref, w_ref, b_ref, out_ref):
    out_ref[...] = x_ref[...] @ w_ref[...] + b_ref[...]


def _tc_encode(x, enc_W, enc_b):
    grid = (NP + NB - 1) // NB
    return pl.pallas_call(
        _enc_body,
        grid=(grid,),
        in_specs=[
            pl.BlockSpec((NB, 3), lambda i: (i, 0)),
            pl.BlockSpec((3, CH), lambda i: (0, 0)),
            pl.BlockSpec((1, CH), lambda i: (0, 0)),
        ],
        out_specs=pl.BlockSpec((NB, CH), lambda i: (i, 0)),
        out_shape=jax.ShapeDtypeStruct((NP, CH), jnp.float32),
    )(x, enc_W, enc_b.reshape(1, CH))


def _mlp_body(xdp_ref, xmp_ref, w0t_ref, b0_ref, w1t_ref, b1_ref, w2t_ref,
              b2_ref, out_ref):
    npc = BE // 128
    xdp = xdp_ref[...]
    xmp = xmp_ref[...]
    xd = jnp.concatenate([xdp[j] for j in range(npc)], axis=1)
    xm = jnp.concatenate([xmp[j] for j in range(npc)], axis=1)
    h = jnp.concatenate([xd, xm], axis=0)
    h = jnp.maximum(w0t_ref[...] @ h + b0_ref[...], 0.0)
    h = jnp.maximum(w1t_ref[...] @ h + b1_ref[...], 0.0)
    m = w2t_ref[...] @ h + b2_ref[...]
    out_ref[...] = jnp.stack(
        [m[:, 128 * j:128 * (j + 1)] for j in range(npc)], axis=0)


def _tc_edge_mlp(xdp, xmp, W0, b0, W1, b1, W2, b2):
    npc = BE // 128
    grid = EP // BE
    return pl.pallas_call(
        _mlp_body,
        grid=(grid,),
        in_specs=[
            pl.BlockSpec((npc, CH, 128), lambda i: (i, 0, 0)),
            pl.BlockSpec((npc, CH, 128), lambda i: (i, 0, 0)),
            pl.BlockSpec((32, 2 * CH), lambda i: (0, 0)),
            pl.BlockSpec((32, 1), lambda i: (0, 0)),
            pl.BlockSpec((32, 32), lambda i: (0, 0)),
            pl.BlockSpec((32, 1), lambda i: (0, 0)),
            pl.BlockSpec((CH, 32), lambda i: (0, 0)),
            pl.BlockSpec((CH, 1), lambda i: (0, 0)),
        ],
        out_specs=pl.BlockSpec((npc, CH, 128), lambda i: (i, 0, 0)),
        out_shape=jax.ShapeDtypeStruct((NPIECE, CH, 128), jnp.float32),
    )(xdp, xmp, W0.T, b0.reshape(32, 1), W1.T, b1.reshape(32, 1),
      W2.T, b2.reshape(CH, 1))


def _combine_body(part_ref, out_ref):
    p = part_ref[...]
    m = jnp.max(p, axis=0)
    m = jnp.where(jnp.isneginf(m), 0.0, m)
    m = jnp.maximum(m, 0.0)
    out_ref[...] = m.T


def _tc_combine(part):
    grid = (NP + NB - 1) // NB
    return pl.pallas_call(
        _combine_body,
        grid=(grid,),
        in_specs=[pl.BlockSpec((NQ, CH, NB), lambda i: (0, 0, i))],
        out_specs=pl.BlockSpec((NB, CH), lambda i: (i, 0)),
        out_shape=jax.ShapeDtypeStruct((NP, CH), jnp.float32),
    )(part)


def _head_body(part_ref, w0_ref, b0_ref, w1_ref, b1_ref, w2_ref, b2_ref,
               out_ref):
    p = part_ref[...]
    m = jnp.max(p, axis=0)
    m = jnp.where(jnp.isneginf(m), 0.0, m)
    m = jnp.maximum(m, 0.0)
    xb = m.T
    h = jnp.maximum(xb @ w0_ref[...] + b0_ref[...], 0.0)
    h = jnp.maximum(h @ w1_ref[...] + b1_ref[...], 0.0)
    out_ref[...] = h @ w2_ref[...] + b2_ref[...]


def _tc_head(part, W0, b0, W1, b1, W2, b2):
    grid = (N + NB - 1) // NB
    return pl.pallas_call(
        _head_body,
        grid=(grid,),
        in_specs=[
            pl.BlockSpec((NQ, CH, NB), lambda i: (0, 0, i)),
            pl.BlockSpec((CH, 128), lambda i: (0, 0)),
            pl.BlockSpec((1, 128), lambda i: (0, 0)),
            pl.BlockSpec((128, 128), lambda i: (0, 0)),
            pl.BlockSpec((1, 128), lambda i: (0, 0)),
            pl.BlockSpec((128, 1), lambda i: (0, 0)),
            pl.BlockSpec((1, 1), lambda i: (0, 0)),
        ],
        out_specs=pl.BlockSpec((NB, 1), lambda i: (i, 0)),
        out_shape=jax.ShapeDtypeStruct((N, 1), jnp.float32),
    )(part, W0, b0.reshape(1, 128), W1, b1.reshape(1, 128),
      W2, b2.reshape(1, 1))


def kernel(x, edge_index, enc_W, enc_b,
           c0_W0, c0_b0, c0_W1, c0_b1, c0_W2, c0_b2,
           c1_W0, c1_b0, c1_W1, c1_b1, c1_W2, c1_b2,
           m_W0, m_b0, m_W1, m_b1, m_W2, m_b2):
    src = edge_index[0].astype(jnp.int32)
    dst = edge_index[1].astype(jnp.int32)
    pad = EP - E
    src = jnp.concatenate([src, jnp.zeros((pad,), jnp.int32)])
    dst = jnp.concatenate([dst, jnp.full((pad,), N, jnp.int32)])

    X = _tc_encode(x, enc_W, enc_b)

    xdp, xmp = _sc_gather(X, dst, src)
    msgp = _tc_edge_mlp(xdp, xmp, c0_W0, c0_b0, c0_W1, c0_b1, c0_W2, c0_b2)
    part = _sc_scatter(msgp, dst)
    X = _tc_combine(part)

    xdp, xmp = _sc_gather(X, dst, src)
    msgp = _tc_edge_mlp(xdp, xmp, c1_W0, c1_b0, c1_W1, c1_b1, c1_W2, c1_b2)
    part = _sc_scatter(msgp, dst)

    return _tc_head(part, m_W0, m_b0, m_W1, m_b1, m_W2, m_b2)


# speedup vs baseline: 1.7261x; 1.2482x over previous
import functools

import jax
import jax.numpy as jnp
from jax import lax
from jax.experimental import pallas as pl
from jax.experimental.pallas import tpu as pltpu
from jax.experimental.pallas import tpu_sc as plsc

N = 100000
E = 3200000
CH = 8

NC = 2
NS = 16
NW = NC * NS
EP = 3211264
NPIECE = EP // 128
EW = EP // NW
CEG = 2048
NCHG = EW // CEG
NQ = 4
EQ = EP // NQ
CES = 2048
NCHS = EQ // CES

BE = 8192
NB = 2048
NP = 100096


def _sc_mesh():
    return plsc.VectorSubcoreMesh(core_axis_name="c", subcore_axis_name="s")


_SC_PARAMS = pltpu.CompilerParams(
    use_tc_tiling_on_sc=False, needs_layout_passes=False
)


def _gather_body(x8_hbm, dst_hbm, src_hbm, xdp_hbm, xmp_hbm,
                 ib0_d, ib0_s, ib1_d, ib1_s,
                 rows0_d, rows0_s, rows1_d, rows1_s,
                 tbuf_d, tbuf_m, sem_i, sem_g, sem_o):
    wid = lax.axis_index("s") * NC + lax.axis_index("c")
    base0 = wid * EW
    lanes = lax.iota(jnp.int32, 16)

    def fire_idx(kc, ib_d, ib_s):
        b = base0 + kc * CEG
        pltpu.async_copy(dst_hbm.at[pl.ds(b, CEG)], ib_d, sem_i)
        pltpu.async_copy(src_hbm.at[pl.ds(b, CEG)], ib_s, sem_i)

    def drain_idx(ib_d, ib_s):
        pltpu.make_async_copy(dst_hbm.at[pl.ds(0, CEG)], ib_d, sem_i).wait()
        pltpu.make_async_copy(src_hbm.at[pl.ds(0, CEG)], ib_s, sem_i).wait()

    def fire_gathers(ib, rows):
        for j in range(CEG // 128):
            pltpu.async_copy(
                x8_hbm.at[ib.at[pl.ds(j * 128, 128)]],
                rows.at[pl.ds(j * 128, 128)], sem_g)

    def drain_gathers(rows):
        pltpu.make_async_copy(x8_hbm.at[pl.ds(0, CEG)], rows, sem_g).wait()

    fire_idx(0, ib0_d, ib0_s)
    drain_idx(ib0_d, ib0_s)
    fire_gathers(ib0_d, rows0_d)
    fire_gathers(ib0_s, rows0_s)

    def do_chunk(k, bufs, nbufs):
        ib_d, ib_s, rows_d, rows_s = bufs
        nib_d, nib_s, nrows_d, nrows_s = nbufs
        kn = jnp.minimum(k + 1, NCHG - 1)
        fire_idx(kn, nib_d, nib_s)
        drain_gathers(rows_d)
        drain_gathers(rows_s)
        drain_idx(nib_d, nib_s)
        fire_gathers(nib_d, nrows_d)
        fire_gathers(nib_s, nrows_s)
        @plsc.parallel_loop(0, CEG // 16, unroll=2)
        def tgrp(g):
            ridx = lanes + g * 16
            pe = g // 8
            off = (g % 8) * 16
            for c in range(CH):
                csp = jnp.full((16,), c, jnp.int32)
                a = plsc.load_gather(rows_d, [ridx, csp])
                b = plsc.load_gather(rows_s, [ridx, csp])
                tbuf_d[pe, c, pl.ds(off, 16)] = a
                tbuf_m[pe, c, pl.ds(off, 16)] = b - a
        pb = (base0 + k * CEG) // 128
        pltpu.sync_copy(tbuf_d, xdp_hbm.at[pl.ds(pb, CEG // 128)])
        pltpu.sync_copy(tbuf_m, xmp_hbm.at[pl.ds(pb, CEG // 128)])

    bufs0 = (ib0_d, ib0_s, rows0_d, rows0_s)
    bufs1 = (ib1_d, ib1_s, rows1_d, rows1_s)

    def pair(jj, carry):
        do_chunk(jj * 2, bufs0, bufs1)
        do_chunk(jj * 2 + 1, bufs1, bufs0)
        return carry

    lax.fori_loop(0, NCHG // 2, pair, 0)
    do_chunk(NCHG - 1, bufs0, bufs1)
    drain_gathers(rows1_d)
    drain_gathers(rows1_s)


def _sc_gather(x8, dst, src):
    k = functools.partial(
        pl.kernel,
        out_type=(
            jax.ShapeDtypeStruct((NPIECE, CH, 128), jnp.float32),
            jax.ShapeDtypeStruct((NPIECE, CH, 128), jnp.float32),
        ),
        mesh=_sc_mesh(),
        scratch_types=[
            pltpu.VMEM((CEG,), jnp.int32),
            pltpu.VMEM((CEG,), jnp.int32),
            pltpu.VMEM((CEG,), jnp.int32),
            pltpu.VMEM((CEG,), jnp.int32),
            pltpu.VMEM((CEG, CH), jnp.float32),
            pltpu.VMEM((CEG, CH), jnp.float32),
            pltpu.VMEM((CEG, CH), jnp.float32),
            pltpu.VMEM((CEG, CH), jnp.float32),
            pltpu.VMEM((CEG // 128, CH, 128), jnp.float32),
            pltpu.VMEM((CEG // 128, CH, 128), jnp.float32),
            pltpu.SemaphoreType.DMA,
            pltpu.SemaphoreType.DMA,
            pltpu.SemaphoreType.DMA,
        ],
        compiler_params=_SC_PARAMS,
    )(_gather_body)
    return k(x8, dst, src)


def _scatter_body_full(msgp_hbm, dst_hbm, part_hbm, acc_v,
                       ib0, vb0, ib1, vb1, sem_i, sem_v):
    wid = lax.axis_index("s") * NC + lax.axis_index("c")
    ch = wid // NQ
    q = wid % NQ

    minf = jnp.full((16,), -jnp.inf, jnp.float32)

    @plsc.parallel_loop(0, NP // 16, unroll=4)
    def initb(i):
        acc_v[pl.ds(i * 16, 16)] = minf

    base0 = q * EQ

    def fire(kc, ib, vb):
        b = base0 + kc * CES
        pltpu.async_copy(dst_hbm.at[pl.ds(b, CES)], ib, sem_i)
        pltpu.async_copy(
            msgp_hbm.at[pl.ds(b // 128, CES // 128), ch], vb, sem_v)

    def drain(ib, vb):
        pltpu.make_async_copy(dst_hbm.at[pl.ds(0, CES)], ib, sem_i).wait()
        pltpu.make_async_copy(
            msgp_hbm.at[pl.ds(0, CES // 128), 0], vb, sem_v).wait()

    fire(0, ib0, vb0)

    def do_chunk(k, ib, vb, nib, nvb):
        kn = jnp.minimum(k + 1, NCHS - 1)
        fire(kn, nib, nvb)
        drain(ib, vb)

        def grp_fast(g, carry):
            idx = ib[pl.ds(g * 16, 16)]
            val = vb[g // 8, pl.ds((g % 8) * 16, 16)]
            cur = plsc.load_gather(acc_v, [idx])
            plsc.store_scatter(acc_v, [idx], jnp.maximum(cur, val))
            return carry

        lax.fori_loop(0, CES // 16, grp_fast, 0, unroll=2)

        half = CES // 32
        for h in range(2):
            g0 = h * half

            @plsc.parallel_loop(
                g0, g0 + half, unroll=4, carry=jnp.zeros((16,), jnp.bool_))
            def flag(g, fl):
                idx = ib[pl.ds(g * 16, 16)]
                val = vb[g // 8, pl.ds((g % 8) * 16, 16)]
                got = plsc.load_gather(acc_v, [idx])
                return fl | (val > got)

            @pl.when(jnp.any(flag))
            def _(g0=g0):
                def grp_slow(g, carry):
                    idx = ib[pl.ds(g * 16, 16)]
                    val = vb[g // 8, pl.ds((g % 8) * 16, 16)]
                    cur = plsc.load_gather(acc_v, [idx])
                    plsc.store_scatter(acc_v, [idx], jnp.maximum(cur, val))
                    got = plsc.load_gather(acc_v, [idx])
                    pend = val > got

                    @pl.when(jnp.any(pend))
                    def _():
                        def fix(i, s):
                            got_, p_ = s
                            plsc.store_scatter(
                                acc_v, [idx], jnp.maximum(got_, val),
                                mask=p_)
                            g2 = plsc.load_gather(acc_v, [idx])
                            return g2, val > g2

                        lax.fori_loop(0, 15, fix, (got, pend))

                    return carry

                lax.fori_loop(g0, g0 + half, grp_slow, 0)

    def pair(jj, carry):
        do_chunk(jj * 2, ib0, vb0, ib1, vb1)
        do_chunk(jj * 2 + 1, ib1, vb1, ib0, vb0)
        return carry

    lax.fori_loop(0, NCHS // 2, pair, 0)
    drain(ib0, vb0)
    pltpu.sync_copy(acc_v, part_hbm.at[q, ch])


def _sc_scatter(msgp, dst):
    k = functools.partial(
        pl.kernel,
        out_type=jax.ShapeDtypeStruct((NQ, CH, NP), jnp.float32),
        mesh=_sc_mesh(),
        scratch_types=[
            pltpu.VMEM((NP,), jnp.float32),
            pltpu.VMEM((CES,), jnp.int32),
            pltpu.VMEM((CES // 128, 128), jnp.float32),
            pltpu.VMEM((CES,), jnp.int32),
            pltpu.VMEM((CES // 128, 128), jnp.float32),
            pltpu.SemaphoreType.DMA,
            pltpu.SemaphoreType.DMA,
        ],
        compiler_params=_SC_PARAMS,
    )(_scatter_body_full)
    return k(msgp, dst)


def _enc_body(x_ref, w_ref, b_ref, out_ref):
    out_ref[...] = x_ref[...] @ w_ref[...] + b_ref[...]


def _tc_encode(x, enc_W, enc_b):
    grid = (NP + NB - 1) // NB
    return pl.pallas_call(
        _enc_body,
        grid=(grid,),
        in_specs=[
            pl.BlockSpec((NB, 3), lambda i: (i, 0)),
            pl.BlockSpec((3, CH), lambda i: (0, 0)),
            pl.BlockSpec((1, CH), lambda i: (0, 0)),
        ],
        out_specs=pl.BlockSpec((NB, CH), lambda i: (i, 0)),
        out_shape=jax.ShapeDtypeStruct((NP, CH), jnp.float32),
    )(x, enc_W, enc_b.reshape(1, CH))


def _mlp_body(xdp_ref, xmp_ref, w0t_ref, b0_ref, w1t_ref, b1_ref, w2t_ref,
              b2_ref, out_ref):
    npc = BE // 128
    xdp = xdp_ref[...]
    xmp = xmp_ref[...]
    xd = jnp.concatenate([xdp[j] for j in range(npc)], axis=1)
    xm = jnp.concatenate([xmp[j] for j in range(npc)], axis=1)
    h = jnp.concatenate([xd, xm], axis=0)
    h = jnp.maximum(w0t_ref[...] @ h + b0_ref[...], 0.0)
    h = jnp.maximum(w1t_ref[...] @ h + b1_ref[...], 0.0)
    m = w2t_ref[...] @ h + b2_ref[...]
    out_ref[...] = jnp.stack(
        [m[:, 128 * j:128 * (j + 1)] for j in range(npc)], axis=0)


def _tc_edge_mlp(xdp, xmp, W0, b0, W1, b1, W2, b2):
    npc = BE // 128
    grid = EP // BE
    return pl.pallas_call(
        _mlp_body,
        grid=(grid,),
        in_specs=[
            pl.BlockSpec((npc, CH, 128), lambda i: (i, 0, 0)),
            pl.BlockSpec((npc, CH, 128), lambda i: (i, 0, 0)),
            pl.BlockSpec((32, 2 * CH), lambda i: (0, 0)),
            pl.BlockSpec((32, 1), lambda i: (0, 0)),
            pl.BlockSpec((32, 32), lambda i: (0, 0)),
            pl.BlockSpec((32, 1), lambda i: (0, 0)),
            pl.BlockSpec((CH, 32), lambda i: (0, 0)),
            pl.BlockSpec((CH, 1), lambda i: (0, 0)),
        ],
        out_specs=pl.BlockSpec((npc, CH, 128), lambda i: (i, 0, 0)),
        out_shape=jax.ShapeDtypeStruct((NPIECE, CH, 128), jnp.float32),
    )(xdp, xmp, W0.T, b0.reshape(32, 1), W1.T, b1.reshape(32, 1),
      W2.T, b2.reshape(CH, 1))


def _combine_body(part_ref, out_ref):
    p = part_ref[...]
    m = jnp.max(p, axis=0)
    m = jnp.where(jnp.isneginf(m), 0.0, m)
    m = jnp.maximum(m, 0.0)
    out_ref[...] = m.T


def _tc_combine(part):
    grid = (NP + NB - 1) // NB
    return pl.pallas_call(
        _combine_body,
        grid=(grid,),
        in_specs=[pl.BlockSpec((NQ, CH, NB), lambda i: (0, 0, i))],
        out_specs=pl.BlockSpec((NB, CH), lambda i: (i, 0)),
        out_shape=jax.ShapeDtypeStruct((NP, CH), jnp.float32),
    )(part)


def _head_body(part_ref, w0_ref, b0_ref, w1_ref, b1_ref, w2_ref, b2_ref,
               out_ref):
    p = part_ref[...]
    m = jnp.max(p, axis=0)
    m = jnp.where(jnp.isneginf(m), 0.0, m)
    m = jnp.maximum(m, 0.0)
    xb = m.T
    h = jnp.maximum(xb @ w0_ref[...] + b0_ref[...], 0.0)
    h = jnp.maximum(h @ w1_ref[...] + b1_ref[...], 0.0)
    out_ref[...] = h @ w2_ref[...] + b2_ref[...]


def _tc_head(part, W0, b0, W1, b1, W2, b2):
    grid = (N + NB - 1) // NB
    return pl.pallas_call(
        _head_body,
        grid=(grid,),
        in_specs=[
            pl.BlockSpec((NQ, CH, NB), lambda i: (0, 0, i)),
            pl.BlockSpec((CH, 128), lambda i: (0, 0)),
            pl.BlockSpec((1, 128), lambda i: (0, 0)),
            pl.BlockSpec((128, 128), lambda i: (0, 0)),
            pl.BlockSpec((1, 128), lambda i: (0, 0)),
            pl.BlockSpec((128, 1), lambda i: (0, 0)),
            pl.BlockSpec((1, 1), lambda i: (0, 0)),
        ],
        out_specs=pl.BlockSpec((NB, 1), lambda i: (i, 0)),
        out_shape=jax.ShapeDtypeStruct((N, 1), jnp.float32),
    )(part, W0, b0.reshape(1, 128), W1, b1.reshape(1, 128),
      W2, b2.reshape(1, 1))


def kernel(x, edge_index, enc_W, enc_b,
           c0_W0, c0_b0, c0_W1, c0_b1, c0_W2, c0_b2,
           c1_W0, c1_b0, c1_W1, c1_b1, c1_W2, c1_b2,
           m_W0, m_b0, m_W1, m_b1, m_W2, m_b2):
    src = edge_index[0].astype(jnp.int32)
    dst = edge_index[1].astype(jnp.int32)
    pad = EP - E
    src = jnp.concatenate([src, jnp.zeros((pad,), jnp.int32)])
    dst = jnp.concatenate([dst, jnp.full((pad,), N, jnp.int32)])

    X = _tc_encode(x, enc_W, enc_b)

    xdp, xmp = _sc_gather(X, dst, src)
    msgp = _tc_edge_mlp(xdp, xmp, c0_W0, c0_b0, c0_W1, c0_b1, c0_W2, c0_b2)
    part = _sc_scatter(msgp, dst)
    X = _tc_combine(part)

    xdp, xmp = _sc_gather(X, dst, src)
    msgp = _tc_edge_mlp(xdp, xmp, c1_W0, c1_b0, c1_W1, c1_b1, c1_W2, c1_b2)
    part = _sc_scatter(msgp, dst)

    return _tc_head(part, m_W0, m_b0, m_W1, m_b1, m_W2, m_b2)
